# baseline (device time: 8132 ns/iter reference)
import jax
import jax.numpy as jnp
from jax import lax
from jax.experimental import pallas as pl
from jax.experimental.pallas import tpu as pltpu

N_CHK = 4


def kernel(x):
    m, n = x.shape
    rows, lanes = m // 128, 128
    bm = m // N_CHK
    brows = bm // 128
    half = rows // 2

    def body(
        x_hbm, out_hbm, xbuf, comm_ref, copy_sems, send_sems, recv_sems,
        out_sem,
    ):
        my_x = lax.axis_index("x")
        my_y = lax.axis_index("y")
        nbr = (my_x, 1 - my_y)

        barrier_sem = pltpu.get_barrier_semaphore()
        pl.semaphore_signal(
            barrier_sem, inc=1, device_id=nbr,
            device_id_type=pl.DeviceIdType.MESH,
        )

        def fetch(k, slot):
            return pltpu.make_async_copy(
                x_hbm.at[pl.ds(k * bm, bm), :],
                xbuf.at[slot],
                copy_sems.at[slot],
            )

        def make_rdma(sl, k):
            return pltpu.make_async_remote_copy(
                src_ref=comm_ref.at[0, sl],
                dst_ref=comm_ref.at[1, sl],
                send_sem=send_sems.at[k],
                recv_sem=recv_sems.at[k],
                device_id=nbr,
                device_id_type=pl.DeviceIdType.MESH,
            )

        fetch(0, 0).start()
        for k in range(N_CHK):
            if k + 1 < N_CHK:
                fetch(k + 1, (k + 1) % 2).start()
            fetch(k, k % 2).wait()
            partial = jnp.max(xbuf[k % 2], axis=1)
            comm_ref[0, pl.ds(k * brows, brows), :] = jnp.reshape(
                partial, (brows, lanes)
            )
            if k == N_CHK // 2 - 1:
                pl.semaphore_wait(barrier_sem, 1)
                make_rdma(pl.ds(0, half), 0).start()

        make_rdma(pl.ds(half, rows - half), 1).start()
        make_rdma(pl.ds(0, half), 0).wait()
        make_rdma(pl.ds(half, rows - half), 1).wait()
        comm_ref[0, :, :] = jnp.maximum(comm_ref[0, :, :], comm_ref[1, :, :])

        store = pltpu.make_async_copy(comm_ref.at[0], out_hbm, out_sem)
        store.start()
        store.wait()

    packed = pl.pallas_call(
        body,
        out_shape=jax.ShapeDtypeStruct((rows, lanes), jnp.float32),
        in_specs=[pl.BlockSpec(memory_space=pltpu.MemorySpace.HBM)],
        out_specs=pl.BlockSpec(memory_space=pltpu.MemorySpace.HBM),
        scratch_shapes=[
            pltpu.VMEM((2, bm, n), jnp.float32),
            pltpu.VMEM((2, rows, lanes), jnp.float32),
            pltpu.SemaphoreType.DMA((2,)),
            pltpu.SemaphoreType.DMA((2,)),
            pltpu.SemaphoreType.DMA((2,)),
            pltpu.SemaphoreType.DMA,
        ],
        compiler_params=pltpu.CompilerParams(collective_id=0),
    )(x)
    return jnp.reshape(packed, (m, 1))


# device time: 7522 ns/iter; 1.0811x vs baseline; 1.0811x over previous
import jax
import jax.numpy as jnp
from jax import lax
from jax.experimental import pallas as pl
from jax.experimental.pallas import tpu as pltpu

N_CHK = 4


def kernel(x):
    m, n = x.shape
    rows, lanes = m // 128, 128
    bm = m // N_CHK
    brows = bm // 128
    half = rows // 2

    def body(
        x_hbm, out_hbm, xbuf, comm_ref, copy_sems, send_sems, recv_sems,
        out_sem,
    ):
        my_x = lax.axis_index("x")
        my_y = lax.axis_index("y")
        nbr = (my_x, 1 - my_y)

        barrier_sem = pltpu.get_barrier_semaphore()
        pl.semaphore_signal(
            barrier_sem, inc=1, device_id=nbr,
            device_id_type=pl.DeviceIdType.MESH,
        )

        def fetch(k, slot):
            return pltpu.make_async_copy(
                x_hbm.at[pl.ds(k * bm, bm), :],
                xbuf.at[slot],
                copy_sems.at[slot],
            )

        def make_rdma(sl, k):
            return pltpu.make_async_remote_copy(
                src_ref=comm_ref.at[0, sl],
                dst_ref=comm_ref.at[1, sl],
                send_sem=send_sems.at[k],
                recv_sem=recv_sems.at[k],
                device_id=nbr,
                device_id_type=pl.DeviceIdType.MESH,
            )

        fetch(0, 0).start()
        for k in range(N_CHK):
            if k + 1 < N_CHK:
                fetch(k + 1, (k + 1) % 2).start()
            fetch(k, k % 2).wait()
            partial = jnp.max(xbuf[k % 2], axis=1)
            comm_ref[0, pl.ds(k * brows, brows), :] = jnp.reshape(
                partial, (brows, lanes)
            )
            if k == N_CHK // 2 - 1:
                pl.semaphore_wait(barrier_sem, 1)
                make_rdma(pl.ds(0, half), 0).start()

        make_rdma(pl.ds(half, rows - half), 1).start()
        make_rdma(pl.ds(0, half), 0).wait()
        make_rdma(pl.ds(half, rows - half), 1).wait()
        comm_ref[0, :, :] = jnp.maximum(comm_ref[0, :, :], comm_ref[1, :, :])

        store = pltpu.make_async_copy(comm_ref.at[0], out_hbm, out_sem)
        store.start()
        store.wait()

    packed = pl.pallas_call(
        body,
        out_shape=jax.ShapeDtypeStruct((rows, lanes), jnp.float32),
        in_specs=[pl.BlockSpec(memory_space=pltpu.MemorySpace.HBM)],
        out_specs=pl.BlockSpec(memory_space=pltpu.MemorySpace.HBM),
        scratch_shapes=[
            pltpu.VMEM((2, bm, n), jnp.float32),
            pltpu.VMEM((2, rows, lanes), jnp.float32),
            pltpu.SemaphoreType.DMA((2,)),
            pltpu.SemaphoreType.DMA((2,)),
            pltpu.SemaphoreType.DMA((2,)),
            pltpu.SemaphoreType.DMA,
        ],
        compiler_params=pltpu.CompilerParams(collective_id=0),
    )(pltpu.with_memory_space_constraint(x, pltpu.MemorySpace.HBM))
    return jnp.reshape(packed, (m, 1))


# device time: 6750 ns/iter; 1.2047x vs baseline; 1.1144x over previous
import jax
import jax.numpy as jnp
from jax import lax
from jax.experimental import pallas as pl
from jax.experimental.pallas import tpu as pltpu

N_CHK = 4


def kernel(x):
    m, n = x.shape
    rows, lanes = m // 128, 128
    bm = m // N_CHK
    brows = bm // 128
    half = rows // 2

    def body(
        x_hbm, out_hbm, xbuf, comm_ref, copy_sems, send_sems, recv_sems,
        out_sem,
    ):
        my_x = lax.axis_index("x")
        my_y = lax.axis_index("y")
        nbr = (my_x, 1 - my_y)

        barrier_sem = pltpu.get_barrier_semaphore()
        pl.semaphore_signal(
            barrier_sem, inc=1, device_id=nbr,
            device_id_type=pl.DeviceIdType.MESH,
        )

        def fetch(k):
            return pltpu.make_async_copy(
                x_hbm.at[pl.ds(k * bm, bm), :],
                xbuf.at[k],
                copy_sems.at[k],
            )

        def make_rdma(sl, k):
            return pltpu.make_async_remote_copy(
                src_ref=comm_ref.at[0, sl],
                dst_ref=comm_ref.at[1, sl],
                send_sem=send_sems.at[k],
                recv_sem=recv_sems.at[k],
                device_id=nbr,
                device_id_type=pl.DeviceIdType.MESH,
            )

        for k in range(N_CHK):
            fetch(k).start()
        for k in range(N_CHK):
            fetch(k).wait()
            partial = jnp.max(xbuf[k], axis=1)
            comm_ref[0, pl.ds(k * brows, brows), :] = jnp.reshape(
                partial, (brows, lanes)
            )
            if k == N_CHK // 2 - 1:
                pl.semaphore_wait(barrier_sem, 1)
                make_rdma(pl.ds(0, half), 0).start()

        make_rdma(pl.ds(half, rows - half), 1).start()
        make_rdma(pl.ds(0, half), 0).wait()
        out_lo = jnp.maximum(
            comm_ref[0, pl.ds(0, half), :], comm_ref[1, pl.ds(0, half), :]
        )
        comm_ref[0, pl.ds(0, half), :] = out_lo
        make_rdma(pl.ds(half, rows - half), 1).wait()
        comm_ref[0, pl.ds(half, rows - half), :] = jnp.maximum(
            comm_ref[0, pl.ds(half, rows - half), :],
            comm_ref[1, pl.ds(half, rows - half), :],
        )

        store = pltpu.make_async_copy(comm_ref.at[0], out_hbm, out_sem)
        store.start()
        store.wait()

    packed = pl.pallas_call(
        body,
        out_shape=jax.ShapeDtypeStruct((rows, lanes), jnp.float32),
        in_specs=[pl.BlockSpec(memory_space=pltpu.MemorySpace.HBM)],
        out_specs=pl.BlockSpec(memory_space=pltpu.MemorySpace.HBM),
        scratch_shapes=[
            pltpu.VMEM((N_CHK, bm, n), jnp.float32),
            pltpu.VMEM((2, rows, lanes), jnp.float32),
            pltpu.SemaphoreType.DMA((N_CHK,)),
            pltpu.SemaphoreType.DMA((2,)),
            pltpu.SemaphoreType.DMA((2,)),
            pltpu.SemaphoreType.DMA,
        ],
        compiler_params=pltpu.CompilerParams(collective_id=0),
    )(pltpu.with_memory_space_constraint(x, pltpu.MemorySpace.HBM))
    return jnp.reshape(packed, (m, 1))
